# Initial kernel scaffold; baseline (speedup 1.0000x reference)
import jax
import jax.numpy as jnp
from jax.experimental import pallas as pl


def kernel(child_feats, child_exists, edge_type_onehot, edge_indices, W_child, b_child, W_ne0, b_ne0, W_ne1, b_ne1, W_parent, b_parent):
    raise NotImplementedError("write your pallas kernel here")


# speedup vs baseline: 3.6420x; 3.6420x over previous
import functools

import jax
import jax.numpy as jnp
from jax import lax
from jax.experimental import pallas as pl
from jax.experimental.pallas import tpu as pltpu
from jax.experimental.pallas import tpu_sc as plsc

N = 10000
E = 320000
D = 128
H = 128
ET = 4

NC = 2
NS = 16
NW = NC * NS
EW = E // NW
K = 80
NCHUNK = EW // K
CHR = 80
NRCH = N // CHR
LANES = 16

BN = 1000
BE = 4000


def _dense0_body(x_ref, ex_ref, wc_ref, bc_ref, wa_ref, wb_ref,
                 a_ref, b_ref, s_ref):
    cf = jnp.maximum(
        jnp.dot(x_ref[...], wc_ref[...], preferred_element_type=jnp.float32)
        + bc_ref[...], 0.0)
    cf = cf * ex_ref[...]
    a_ref[...] = jnp.dot(cf, wa_ref[...], preferred_element_type=jnp.float32)
    b_ref[...] = jnp.dot(cf, wb_ref[...], preferred_element_type=jnp.float32)

    @pl.when(pl.program_id(0) == 0)
    def _():
        s_ref[...] = jnp.zeros_like(s_ref)
    s_ref[...] += jnp.sum(cf, axis=0, keepdims=True)


def _dense0(x, ex, wc, bc, wa, wb):
    return pl.pallas_call(
        _dense0_body,
        grid=(N // BN,),
        in_specs=[
            pl.BlockSpec((BN, D), lambda i: (i, 0)),
            pl.BlockSpec((BN, 1), lambda i: (i, 0)),
            pl.BlockSpec((D, H), lambda i: (0, 0)),
            pl.BlockSpec((1, H), lambda i: (0, 0)),
            pl.BlockSpec((H, H), lambda i: (0, 0)),
            pl.BlockSpec((H, H), lambda i: (0, 0)),
        ],
        out_specs=[
            pl.BlockSpec((BN, H), lambda i: (i, 0)),
            pl.BlockSpec((BN, H), lambda i: (i, 0)),
            pl.BlockSpec((1, H), lambda i: (0, 0)),
        ],
        out_shape=[
            jax.ShapeDtypeStruct((N, H), jnp.float32),
            jax.ShapeDtypeStruct((N, H), jnp.float32),
            jax.ShapeDtypeStruct((1, H), jnp.float32),
        ],
    )(x, ex, wc, bc, wa, wb)


def _edgeC_body(ef_ref, w0_ref, b0_ref, w1_ref, b1_ref, c0_ref, c1_ref):
    ef = ef_ref[...]
    c0_ref[...] = (jnp.dot(ef, w0_ref[...], preferred_element_type=jnp.float32)
                   + b0_ref[...])
    c1_ref[...] = (jnp.dot(ef, w1_ref[...], preferred_element_type=jnp.float32)
                   + b1_ref[...])


def _edgeC(ef, w0, b0, w1, b1):
    return pl.pallas_call(
        _edgeC_body,
        grid=(E // BE,),
        in_specs=[
            pl.BlockSpec((BE, ET), lambda i: (i, 0)),
            pl.BlockSpec((ET, H), lambda i: (0, 0)),
            pl.BlockSpec((1, H), lambda i: (0, 0)),
            pl.BlockSpec((ET, H), lambda i: (0, 0)),
            pl.BlockSpec((1, H), lambda i: (0, 0)),
        ],
        out_specs=[
            pl.BlockSpec((BE, H), lambda i: (i, 0)),
            pl.BlockSpec((BE, H), lambda i: (i, 0)),
        ],
        out_shape=[
            jax.ShapeDtypeStruct((E, H), jnp.float32),
            jax.ShapeDtypeStruct((E, H), jnp.float32),
        ],
    )(ef, w0, b0, w1, b1)


@functools.cache
def _sc_scatter_kernel():
    return pl.kernel(
        _sc_scatter_body,
        out_type=jax.ShapeDtypeStruct((NC, N, H), jnp.float32),
        mesh=plsc.VectorSubcoreMesh(core_axis_name="c", subcore_axis_name="s"),
        scratch_types=[
            pltpu.VMEM_SHARED((N, H), jnp.float32),
            pltpu.VMEM((K,), jnp.int32),
            pltpu.VMEM((K,), jnp.int32),
            pltpu.VMEM((K, H), jnp.float32),
            pltpu.VMEM((K, H), jnp.float32),
            pltpu.VMEM((K, H), jnp.float32),
            pltpu.VMEM((CHR, H), jnp.float32),
            pltpu.SemaphoreType.DMA,
            pltpu.SemaphoreType.DMA,
            pltpu.SemaphoreType.DMA,
        ],
    )


def _sc_scatter(a0, b0, c0, src, dst):
    return _sc_scatter_kernel()(a0, b0, c0, src, dst)


def _sc_scatter_body(a_hbm, b_hbm, c_hbm, src_hbm, dst_hbm, out_hbm,
                acc_sh, sidx, didx, ra, rb, rc, zbuf, sem_a, sem_b, sem_c):
    c = lax.axis_index("c")
    s = lax.axis_index("s")
    wid = c * NS + s

    def zrow(i, _):
        for j in range(H // LANES):
            zbuf[i, pl.ds(j * LANES, LANES)] = jnp.zeros((LANES,), jnp.float32)
        return 0
    lax.fori_loop(0, CHR, zrow, 0)

    def zcp(k, _):
        cid = s + k * NS

        @pl.when(cid < NRCH)
        def _():
            pltpu.sync_copy(zbuf, acc_sh.at[pl.ds(cid * CHR, CHR)])
        return 0
    lax.fori_loop(0, pl.cdiv(NRCH, NS), zcp, 0)
    plsc.subcore_barrier()

    base0 = wid * EW

    def chunk(t, _):
        base = base0 + t * K
        pltpu.sync_copy(src_hbm.at[pl.ds(base, K)], sidx)
        pltpu.sync_copy(dst_hbm.at[pl.ds(base, K)], didx)
        cp_a = pltpu.async_copy(a_hbm.at[sidx], ra, sem_a)
        cp_b = pltpu.async_copy(b_hbm.at[didx], rb, sem_b)
        cp_c = pltpu.async_copy(c_hbm.at[pl.ds(base, K)], rc, sem_c)
        cp_a.wait()
        cp_b.wait()
        cp_c.wait()

        def erow(i, _):
            for j in range(H // LANES):
                sl = pl.ds(j * LANES, LANES)
                ra[i, sl] = jnp.maximum(ra[i, sl] + rb[i, sl] + rc[i, sl], 0.0)
            return 0
        lax.fori_loop(0, K, erow, 0)

        pltpu.sync_copy(ra, acc_sh.at[sidx], add=True)
        return 0
    lax.fori_loop(0, NCHUNK, chunk, 0)
    plsc.subcore_barrier()

    def ocp(k, _):
        cid = s + k * NS

        @pl.when(cid < NRCH)
        def _():
            pltpu.sync_copy(acc_sh.at[pl.ds(cid * CHR, CHR)],
                            out_hbm.at[c, pl.ds(cid * CHR, CHR)])
        return 0
    lax.fori_loop(0, pl.cdiv(NRCH, NS), ocp, 0)


def _mid_body(p_ref, wa_ref, wb_ref, a_ref, b_ref, s_ref):
    cf = p_ref[0] + p_ref[1]
    a_ref[...] = jnp.dot(cf, wa_ref[...], preferred_element_type=jnp.float32)
    b_ref[...] = jnp.dot(cf, wb_ref[...], preferred_element_type=jnp.float32)

    @pl.when(pl.program_id(0) == 0)
    def _():
        s_ref[...] = jnp.zeros_like(s_ref)
    s_ref[...] += jnp.sum(cf, axis=0, keepdims=True)


def _mid(p, wa, wb):
    return pl.pallas_call(
        _mid_body,
        grid=(N // BN,),
        in_specs=[
            pl.BlockSpec((NC, BN, H), lambda i: (0, i, 0)),
            pl.BlockSpec((H, H), lambda i: (0, 0)),
            pl.BlockSpec((H, H), lambda i: (0, 0)),
        ],
        out_specs=[
            pl.BlockSpec((BN, H), lambda i: (i, 0)),
            pl.BlockSpec((BN, H), lambda i: (i, 0)),
            pl.BlockSpec((1, H), lambda i: (0, 0)),
        ],
        out_shape=[
            jax.ShapeDtypeStruct((N, H), jnp.float32),
            jax.ShapeDtypeStruct((N, H), jnp.float32),
            jax.ShapeDtypeStruct((1, H), jnp.float32),
        ],
    )(p, wa, wb)


@functools.cache
def _sc_reduce_kernel():
    return pl.kernel(
        _sc_reduce_body,
        out_type=jax.ShapeDtypeStruct((NW, H), jnp.float32),
        mesh=plsc.VectorSubcoreMesh(core_axis_name="c", subcore_axis_name="s"),
        scratch_types=[
            pltpu.VMEM((K,), jnp.int32),
            pltpu.VMEM((K,), jnp.int32),
            pltpu.VMEM((K, H), jnp.float32),
            pltpu.VMEM((K, H), jnp.float32),
            pltpu.VMEM((K, H), jnp.float32),
            pltpu.VMEM((H,), jnp.float32),
            pltpu.SemaphoreType.DMA,
            pltpu.SemaphoreType.DMA,
            pltpu.SemaphoreType.DMA,
        ],
    )


def _sc_reduce(a1, b1, c1, src, dst):
    return _sc_reduce_kernel()(a1, b1, c1, src, dst)


def _sc_reduce_body(a_hbm, b_hbm, c_hbm, src_hbm, dst_hbm, out_hbm,
               sidx, didx, ra, rb, rc, sbuf, sem_a, sem_b, sem_c):
    c = lax.axis_index("c")
    s = lax.axis_index("s")
    wid = c * NS + s
    base0 = wid * EW

    def chunk(t, acc):
        base = base0 + t * K
        pltpu.sync_copy(src_hbm.at[pl.ds(base, K)], sidx)
        pltpu.sync_copy(dst_hbm.at[pl.ds(base, K)], didx)
        cp_a = pltpu.async_copy(a_hbm.at[sidx], ra, sem_a)
        cp_b = pltpu.async_copy(b_hbm.at[didx], rb, sem_b)
        cp_c = pltpu.async_copy(c_hbm.at[pl.ds(base, K)], rc, sem_c)
        cp_a.wait()
        cp_b.wait()
        cp_c.wait()

        def erow(i, acc):
            new = []
            for j in range(H // LANES):
                sl = pl.ds(j * LANES, LANES)
                new.append(acc[j] + jnp.maximum(ra[i, sl] + rb[i, sl]
                                                + rc[i, sl], 0.0))
            return tuple(new)
        return lax.fori_loop(0, K, erow, acc)

    acc0 = tuple(jnp.zeros((LANES,), jnp.float32) for _ in range(H // LANES))
    acc = lax.fori_loop(0, NCHUNK, chunk, acc0)
    for j in range(H // LANES):
        sbuf[pl.ds(j * LANES, LANES)] = acc[j]
    pltpu.sync_copy(sbuf, out_hbm.at[wid])


def _post_body(s0_ref, s1_ref, s2p_ref, wp0_ref, wp1_ref, wp2_ref, bp_ref,
               o_ref):
    s2 = jnp.sum(s2p_ref[...], axis=0, keepdims=True)
    acc = (jnp.dot(s0_ref[...], wp0_ref[...], preferred_element_type=jnp.float32)
           + jnp.dot(s1_ref[...], wp1_ref[...], preferred_element_type=jnp.float32)
           + jnp.dot(s2, wp2_ref[...], preferred_element_type=jnp.float32)
           + bp_ref[...])
    o_ref[...] = jnp.maximum(acc, 0.0)


def _post(s0, s1, s2p, wp0, wp1, wp2, bp):
    return pl.pallas_call(
        _post_body,
        out_shape=jax.ShapeDtypeStruct((1, D), jnp.float32),
    )(s0, s1, s2p, wp0, wp1, wp2, bp)


def kernel(child_feats, child_exists, edge_type_onehot, edge_indices,
           W_child, b_child, W_ne0, b_ne0, W_ne1, b_ne1, W_parent, b_parent):
    x = child_feats[0]
    ex = child_exists[0]
    ef = edge_type_onehot[0]
    src = edge_indices[0, :, 0]
    dst = edge_indices[0, :, 1]

    w0a, w0b, w0c = W_ne0[:H], W_ne0[H:2 * H], W_ne0[2 * H:]
    w1a, w1b, w1c = W_ne1[:H], W_ne1[H:2 * H], W_ne1[2 * H:]
    wp0, wp1, wp2 = W_parent[:H], W_parent[H:2 * H], W_parent[2 * H:]
    bc = b_child[None]
    b0 = b_ne0[None]
    b1 = b_ne1[None]
    bp = b_parent[None]

    a0, b0rows, s0 = _dense0(x, ex, W_child, bc, w0a, w0b)
    c0, c1 = _edgeC(ef, w0c, b0, w1c, b1)
    p = _sc_scatter(a0, b0rows, c0, src, dst)
    a1, b1rows, s1 = _mid(p, w1a, w1b)
    s2p = _sc_reduce(a1, b1rows, c1, src, dst)
    return _post(s0, s1, s2p, wp0, wp1, wp2, bp)


# speedup vs baseline: 6.0407x; 1.6586x over previous
import functools

import jax
import jax.numpy as jnp
from jax import lax
from jax.experimental import pallas as pl
from jax.experimental.pallas import tpu as pltpu
from jax.experimental.pallas import tpu_sc as plsc

N = 10000
E = 320000
D = 128
H = 128
ET = 4

NC = 2
NS = 16
NW = NC * NS
EW = E // NW
KS = 40
NCHS = EW // KS
KR = 80
NCHR = EW // KR
CHR = 40
NRCH = N // CHR
LANES = 16

BN = 1000
BE = 4000


def _dense0_body(x_ref, ex_ref, wc_ref, bc_ref, wa_ref, wb_ref,
                 a_ref, b_ref, s_ref):
    cf = jnp.maximum(
        jnp.dot(x_ref[...], wc_ref[...], preferred_element_type=jnp.float32)
        + bc_ref[...], 0.0)
    cf = cf * ex_ref[...]
    a_ref[...] = jnp.dot(cf, wa_ref[...], preferred_element_type=jnp.float32)
    b_ref[...] = jnp.dot(cf, wb_ref[...], preferred_element_type=jnp.float32)

    @pl.when(pl.program_id(0) == 0)
    def _():
        s_ref[...] = jnp.zeros_like(s_ref)
    s_ref[...] += jnp.sum(cf, axis=0, keepdims=True)


def _dense0(x, ex, wc, bc, wa, wb):
    return pl.pallas_call(
        _dense0_body,
        grid=(N // BN,),
        in_specs=[
            pl.BlockSpec((BN, D), lambda i: (i, 0)),
            pl.BlockSpec((BN, 1), lambda i: (i, 0)),
            pl.BlockSpec((D, H), lambda i: (0, 0)),
            pl.BlockSpec((1, H), lambda i: (0, 0)),
            pl.BlockSpec((H, H), lambda i: (0, 0)),
            pl.BlockSpec((H, H), lambda i: (0, 0)),
        ],
        out_specs=[
            pl.BlockSpec((BN, H), lambda i: (i, 0)),
            pl.BlockSpec((BN, H), lambda i: (i, 0)),
            pl.BlockSpec((1, H), lambda i: (0, 0)),
        ],
        out_shape=[
            jax.ShapeDtypeStruct((N, H), jnp.float32),
            jax.ShapeDtypeStruct((N, H), jnp.float32),
            jax.ShapeDtypeStruct((1, H), jnp.float32),
        ],
    )(x, ex, wc, bc, wa, wb)


def _edgeC_body(ef_ref, w0_ref, b0_ref, w1_ref, b1_ref, c0_ref, c1_ref):
    ef = ef_ref[...]
    c0_ref[...] = (jnp.dot(ef, w0_ref[...], preferred_element_type=jnp.float32)
                   + b0_ref[...])
    c1_ref[...] = (jnp.dot(ef, w1_ref[...], preferred_element_type=jnp.float32)
                   + b1_ref[...])


def _edgeC(ef, w0, b0, w1, b1):
    return pl.pallas_call(
        _edgeC_body,
        grid=(E // BE,),
        in_specs=[
            pl.BlockSpec((BE, ET), lambda i: (i, 0)),
            pl.BlockSpec((ET, H), lambda i: (0, 0)),
            pl.BlockSpec((1, H), lambda i: (0, 0)),
            pl.BlockSpec((ET, H), lambda i: (0, 0)),
            pl.BlockSpec((1, H), lambda i: (0, 0)),
        ],
        out_specs=[
            pl.BlockSpec((BE, H), lambda i: (i, 0)),
            pl.BlockSpec((BE, H), lambda i: (i, 0)),
        ],
        out_shape=[
            jax.ShapeDtypeStruct((E, H), jnp.float32),
            jax.ShapeDtypeStruct((E, H), jnp.float32),
        ],
    )(ef, w0, b0, w1, b1)


@functools.cache
def _sc_scatter_kernel():
    return pl.kernel(
        _sc_scatter_body,
        out_type=jax.ShapeDtypeStruct((NC, N, H), jnp.float32),
        mesh=plsc.VectorSubcoreMesh(core_axis_name="c", subcore_axis_name="s"),
        scratch_types=[
            pltpu.VMEM_SHARED((N, H), jnp.float32),
            pltpu.VMEM((EW,), jnp.int32),
            pltpu.VMEM((EW,), jnp.int32),
            pltpu.VMEM((KS,), jnp.int32),
            pltpu.VMEM((KS, H), jnp.float32),
            pltpu.VMEM((KS, H), jnp.float32),
            pltpu.VMEM((KS, H), jnp.float32),
            pltpu.VMEM((KS, H), jnp.float32),
            pltpu.VMEM((KS, H), jnp.float32),
            pltpu.VMEM((KS, H), jnp.float32),
            pltpu.SemaphoreType.DMA,
            pltpu.SemaphoreType.DMA,
            pltpu.SemaphoreType.DMA,
            pltpu.SemaphoreType.DMA,
            pltpu.SemaphoreType.DMA,
            pltpu.SemaphoreType.DMA,
        ],
    )


def _sc_scatter(a0, b0, c0, src, dst):
    return _sc_scatter_kernel()(a0, b0, c0, src, dst)


def _sc_scatter_body(a_hbm, b_hbm, c_hbm, src_hbm, dst_hbm, out_hbm,
                     acc_sh, sidx, didx, sidx_w, ra0, rb0, rc0, ra1, rb1, rc1,
                     sa0, sb0, sc0, sa1, sb1, sc1):
    c = lax.axis_index("c")
    s = lax.axis_index("s")
    wid = c * NS + s
    base0 = wid * EW
    bufs = ((ra0, rb0, rc0, sa0, sb0, sc0), (ra1, rb1, rc1, sa1, sb1, sc1))

    pltpu.sync_copy(src_hbm.at[wid], sidx)
    pltpu.sync_copy(dst_hbm.at[wid], didx)

    def zrow(i, _):
        for j in range(H // LANES):
            ra0[i, pl.ds(j * LANES, LANES)] = jnp.zeros((LANES,), jnp.float32)
        return 0
    lax.fori_loop(0, CHR, zrow, 0)

    def zcp(k, _):
        cid = s + k * NS

        @pl.when(cid < NRCH)
        def _():
            pltpu.sync_copy(ra0, acc_sh.at[pl.ds(cid * CHR, CHR)])
        return 0
    lax.fori_loop(0, pl.cdiv(NRCH, NS), zcp, 0)
    plsc.subcore_barrier()

    def issue(t, bi):
        ra, rb, rc, sa, sb, sc_ = bufs[bi]
        pltpu.async_copy(a_hbm.at[sidx.at[pl.ds(t * KS, KS)]], ra, sa)
        pltpu.async_copy(b_hbm.at[didx.at[pl.ds(t * KS, KS)]], rb, sb)
        pltpu.async_copy(c_hbm.at[pl.ds(base0 + t * KS, KS)], rc, sc_)

    def process(t, bi):
        ra, rb, rc, sa, sb, sc_ = bufs[bi]
        pltpu.make_async_copy(a_hbm.at[sidx.at[pl.ds(t * KS, KS)]], ra,
                              sa).wait()
        pltpu.make_async_copy(b_hbm.at[didx.at[pl.ds(t * KS, KS)]], rb,
                              sb).wait()
        pltpu.make_async_copy(c_hbm.at[pl.ds(base0 + t * KS, KS)], rc,
                              sc_).wait()

        def erow(i, _):
            for u in range(2):
                for j in range(H // LANES):
                    sl = pl.ds(j * LANES, LANES)
                    r = 2 * i + u
                    ra[r, sl] = jnp.maximum(
                        ra[r, sl] + rb[r, sl] + rc[r, sl], 0.0)
            return 0
        lax.fori_loop(0, KS // 2, erow, 0)

        for o in (0, LANES, KS - LANES):
            sidx_w[pl.ds(o, LANES)] = sidx[pl.ds(t * KS + o, LANES)]
        pltpu.sync_copy(ra, acc_sh.at[sidx_w], add=True)

    issue(0, 0)
    issue(1, 1)

    def pair(t2, _):
        t0 = 2 * t2
        process(t0, 0)

        @pl.when(t0 + 2 < NCHS)
        def _():
            issue(t0 + 2, 0)
        process(t0 + 1, 1)

        @pl.when(t0 + 3 < NCHS)
        def _():
            issue(t0 + 3, 1)
        return 0
    lax.fori_loop(0, NCHS // 2, pair, 0)
    plsc.subcore_barrier()

    def ocp(k, _):
        cid = s + k * NS

        @pl.when(cid < NRCH)
        def _():
            pltpu.sync_copy(acc_sh.at[pl.ds(cid * CHR, CHR)],
                            out_hbm.at[c, pl.ds(cid * CHR, CHR)])
        return 0
    lax.fori_loop(0, pl.cdiv(NRCH, NS), ocp, 0)


def _mid_body(p_ref, wa_ref, wb_ref, a_ref, b_ref, s_ref):
    cf = p_ref[0] + p_ref[1]
    a_ref[...] = jnp.dot(cf, wa_ref[...], preferred_element_type=jnp.float32)
    b_ref[...] = jnp.dot(cf, wb_ref[...], preferred_element_type=jnp.float32)

    @pl.when(pl.program_id(0) == 0)
    def _():
        s_ref[...] = jnp.zeros_like(s_ref)
    s_ref[...] += jnp.sum(cf, axis=0, keepdims=True)


def _mid(p, wa, wb):
    return pl.pallas_call(
        _mid_body,
        grid=(N // BN,),
        in_specs=[
            pl.BlockSpec((NC, BN, H), lambda i: (0, i, 0)),
            pl.BlockSpec((H, H), lambda i: (0, 0)),
            pl.BlockSpec((H, H), lambda i: (0, 0)),
        ],
        out_specs=[
            pl.BlockSpec((BN, H), lambda i: (i, 0)),
            pl.BlockSpec((BN, H), lambda i: (i, 0)),
            pl.BlockSpec((1, H), lambda i: (0, 0)),
        ],
        out_shape=[
            jax.ShapeDtypeStruct((N, H), jnp.float32),
            jax.ShapeDtypeStruct((N, H), jnp.float32),
            jax.ShapeDtypeStruct((1, H), jnp.float32),
        ],
    )(p, wa, wb)


@functools.cache
def _sc_reduce_kernel():
    return pl.kernel(
        _sc_reduce_body,
        out_type=jax.ShapeDtypeStruct((NW, H), jnp.float32),
        mesh=plsc.VectorSubcoreMesh(core_axis_name="c", subcore_axis_name="s"),
        scratch_types=[
            pltpu.VMEM((NCHR, KR), jnp.int32),
            pltpu.VMEM((NCHR, KR), jnp.int32),
            pltpu.VMEM((KR, H), jnp.float32),
            pltpu.VMEM((KR, H), jnp.float32),
            pltpu.VMEM((KR, H), jnp.float32),
            pltpu.VMEM((KR, H), jnp.float32),
            pltpu.VMEM((KR, H), jnp.float32),
            pltpu.VMEM((KR, H), jnp.float32),
            pltpu.VMEM((H,), jnp.float32),
            pltpu.SemaphoreType.DMA,
            pltpu.SemaphoreType.DMA,
            pltpu.SemaphoreType.DMA,
            pltpu.SemaphoreType.DMA,
            pltpu.SemaphoreType.DMA,
            pltpu.SemaphoreType.DMA,
        ],
    )


def _sc_reduce(a1, b1, c1, src, dst):
    return _sc_reduce_kernel()(a1, b1, c1, src, dst)


def _sc_reduce_body(a_hbm, b_hbm, c_hbm, src_hbm, dst_hbm, out_hbm,
                    sidx, didx, ra0, rb0, rc0, ra1, rb1, rc1, sbuf,
                    sa0, sb0, sc0, sa1, sb1, sc1):
    c = lax.axis_index("c")
    s = lax.axis_index("s")
    wid = c * NS + s
    base0 = wid * EW
    bufs = ((ra0, rb0, rc0, sa0, sb0, sc0), (ra1, rb1, rc1, sa1, sb1, sc1))

    pltpu.sync_copy(src_hbm.at[wid], sidx)
    pltpu.sync_copy(dst_hbm.at[wid], didx)

    def issue(t, bi):
        ra, rb, rc, sa, sb, sc_ = bufs[bi]
        pltpu.async_copy(a_hbm.at[sidx.at[t]], ra, sa)
        pltpu.async_copy(b_hbm.at[didx.at[t]], rb, sb)
        pltpu.async_copy(c_hbm.at[pl.ds(base0 + t * KR, KR)], rc, sc_)

    def process(t, bi, acc):
        ra, rb, rc, sa, sb, sc_ = bufs[bi]
        pltpu.make_async_copy(a_hbm.at[sidx.at[t]], ra, sa).wait()
        pltpu.make_async_copy(b_hbm.at[didx.at[t]], rb, sb).wait()
        pltpu.make_async_copy(c_hbm.at[pl.ds(base0 + t * KR, KR)], rc, sc_).wait()

        def erow(i, acc):
            new = list(acc)
            for u in range(2):
                for j in range(H // LANES):
                    sl = pl.ds(j * LANES, LANES)
                    r = 2 * i + u
                    new[j] = new[j] + jnp.maximum(
                        ra[r, sl] + rb[r, sl] + rc[r, sl], 0.0)
            return tuple(new)
        return lax.fori_loop(0, KR // 2, erow, acc)

    issue(0, 0)
    issue(1, 1)
    acc0 = tuple(jnp.zeros((LANES,), jnp.float32) for _ in range(H // LANES))

    def pair(t2, acc):
        t0 = 2 * t2
        acc = process(t0, 0, acc)
        issue(t0 + 2, 0)
        acc = process(t0 + 1, 1, acc)

        @pl.when(t0 + 3 < NCHR)
        def _():
            issue(t0 + 3, 1)
        return acc
    acc = lax.fori_loop(0, (NCHR - 1) // 2, pair, acc0)
    acc = process(NCHR - 1, 0, acc)
    for j in range(H // LANES):
        sbuf[pl.ds(j * LANES, LANES)] = acc[j]
    pltpu.sync_copy(sbuf, out_hbm.at[wid])


def _post_body(s0_ref, s1_ref, s2p_ref, wp0_ref, wp1_ref, wp2_ref, bp_ref,
               o_ref):
    s2 = jnp.sum(s2p_ref[...], axis=0, keepdims=True)
    acc = (jnp.dot(s0_ref[...], wp0_ref[...], preferred_element_type=jnp.float32)
           + jnp.dot(s1_ref[...], wp1_ref[...], preferred_element_type=jnp.float32)
           + jnp.dot(s2, wp2_ref[...], preferred_element_type=jnp.float32)
           + bp_ref[...])
    o_ref[...] = jnp.maximum(acc, 0.0)


def _post(s0, s1, s2p, wp0, wp1, wp2, bp):
    return pl.pallas_call(
        _post_body,
        out_shape=jax.ShapeDtypeStruct((1, D), jnp.float32),
    )(s0, s1, s2p, wp0, wp1, wp2, bp)


def kernel(child_feats, child_exists, edge_type_onehot, edge_indices,
           W_child, b_child, W_ne0, b_ne0, W_ne1, b_ne1, W_parent, b_parent):
    x = child_feats[0]
    ex = child_exists[0]
    ef = edge_type_onehot[0]
    src = edge_indices[0, :, 0]
    dst = edge_indices[0, :, 1]
    src_s = src.reshape(NW, EW)
    dst_s = dst.reshape(NW, EW)
    src_r = src.reshape(NW, NCHR, KR)
    dst_r = dst.reshape(NW, NCHR, KR)

    w0a, w0b, w0c = W_ne0[:H], W_ne0[H:2 * H], W_ne0[2 * H:]
    w1a, w1b, w1c = W_ne1[:H], W_ne1[H:2 * H], W_ne1[2 * H:]
    wp0, wp1, wp2 = W_parent[:H], W_parent[H:2 * H], W_parent[2 * H:]
    bc = b_child[None]
    b0 = b_ne0[None]
    b1 = b_ne1[None]
    bp = b_parent[None]

    a0, b0rows, s0 = _dense0(x, ex, W_child, bc, w0a, w0b)
    c0, c1 = _edgeC(ef, w0c, b0, w1c, b1)
    p = _sc_scatter(a0, b0rows, c0, src_s, dst_s)
    a1, b1rows, s1 = _mid(p, w1a, w1b)
    s2p = _sc_reduce(a1, b1rows, c1, src_r, dst_r)
    return _post(s0, s1, s2p, wp0, wp1, wp2, bp)
